# speedup vs baseline: 1.9369x; 1.9369x over previous
import jax
import jax.numpy as jnp
from jax.experimental import pallas as pl


def _add_kernel(x_ref, pos_ref, out_ref):
    out_ref[...] = x_ref[...] + pos_ref[...][None, :, :]


def kernel(x, pos_table):
    B, L, D = x.shape
    BL = 512
    grid = (L // BL,)
    return pl.pallas_call(
        _add_kernel,
        grid=grid,
        in_specs=[
            pl.BlockSpec((B, BL, D), lambda l: (0, l, 0)),
            pl.BlockSpec((BL, D), lambda l: (l, 0)),
        ],
        out_specs=pl.BlockSpec((B, BL, D), lambda l: (0, l, 0)),
        out_shape=jax.ShapeDtypeStruct((B, L, D), x.dtype),
    )(x, pos_table)
